# baseline (speedup 1.0000x reference)
import functools
import math

import jax
import jax.numpy as jnp
import numpy as np
from jax import lax
from jax.experimental import pallas as pl
from jax.experimental.pallas import tpu as pltpu

D_MODEL = 512
MAX_LEN = 512


def _pe_table_ext() -> np.ndarray:
    pe = np.zeros((MAX_LEN, D_MODEL), dtype=np.float32)
    position = np.arange(0, MAX_LEN, dtype=np.float32)[:, None]
    div_term = np.exp(
        np.arange(0, D_MODEL, 2, dtype=np.float32) * -(math.log(10000.0) / D_MODEL)
    )
    pe[:, 0::2] = np.sin(position * div_term)
    pe[:, 1::2] = np.cos(position * div_term)
    return np.concatenate([np.zeros((1, D_MODEL), np.float32), pe], axis=0)


def _gather_indices(t_lens, D) -> np.ndarray:
    parts = []
    for t in t_lens:
        parts.append(np.zeros((1,), np.int32))
        parts.append(np.linspace(0, D - 1, t).astype(np.int32) + 1)
    return np.concatenate(parts)


def kernel(modal_feat_0, modal_feat_1, modal_feat_2):
    modal_feats = (modal_feat_0, modal_feat_1, modal_feat_2)
    batch = modal_feats[0].shape[0]
    D = modal_feats[0].shape[1] - 1
    t_lens = [m.shape[1] - 1 for m in modal_feats]
    seq = sum(t_lens) + len(t_lens)

    table = _pe_table_ext()
    idx = _gather_indices(t_lens, D)
    nrows = table.shape[0]
    onehot = np.zeros((seq, nrows), np.float32)
    onehot[np.arange(seq), idx] = 1.0

    def body(oh_ref, tab_ref, o_ref, temp):
        @pl.when(pl.program_id(0) == 0)
        def _():
            temp[...] = jnp.dot(
                oh_ref[...], tab_ref[...], preferred_element_type=jnp.float32
            )

        o_ref[...] = temp[...][None]

    return pl.pallas_call(
        body,
        grid=(batch,),
        in_specs=[
            pl.BlockSpec((seq, nrows), lambda b: (0, 0)),
            pl.BlockSpec((nrows, D_MODEL), lambda b: (0, 0)),
        ],
        out_specs=pl.BlockSpec((1, seq, D_MODEL), lambda b: (b, 0, 0)),
        out_shape=jax.ShapeDtypeStruct((batch, seq, D_MODEL), jnp.float32),
        scratch_shapes=[
            pltpu.VMEM((seq, D_MODEL), jnp.float32),
        ],
    )(jnp.asarray(onehot), jnp.asarray(table))


# speedup vs baseline: 1.0389x; 1.0389x over previous
import math

import jax
import jax.numpy as jnp
import numpy as np
from jax.experimental import pallas as pl
from jax.experimental.pallas import tpu as pltpu

D_MODEL = 512
MAX_LEN = 512
NBUF = 4


def _pe_table_ext() -> np.ndarray:
    pe = np.zeros((MAX_LEN, D_MODEL), dtype=np.float32)
    position = np.arange(0, MAX_LEN, dtype=np.float32)[:, None]
    div_term = np.exp(
        np.arange(0, D_MODEL, 2, dtype=np.float32) * -(math.log(10000.0) / D_MODEL)
    )
    pe[:, 0::2] = np.sin(position * div_term)
    pe[:, 1::2] = np.cos(position * div_term)
    return np.concatenate([np.zeros((1, D_MODEL), np.float32), pe], axis=0)


def _gather_indices(t_lens, D) -> np.ndarray:
    parts = []
    for t in t_lens:
        parts.append(np.zeros((1,), np.int32))
        parts.append(np.linspace(0, D - 1, t).astype(np.int32) + 1)
    return np.concatenate(parts)


def kernel(modal_feat_0, modal_feat_1, modal_feat_2):
    modal_feats = (modal_feat_0, modal_feat_1, modal_feat_2)
    batch = modal_feats[0].shape[0]
    D = modal_feats[0].shape[1] - 1
    t_lens = [m.shape[1] - 1 for m in modal_feats]
    seq = sum(t_lens) + len(t_lens)

    table = _pe_table_ext()
    idx = _gather_indices(t_lens, D)
    nrows = table.shape[0]
    onehot = np.zeros((seq, nrows), np.float32)
    onehot[np.arange(seq), idx] = 1.0

    def body(oh_ref, tab_ref, o_ref, *rest):
        temps, sems = rest[:NBUF], rest[NBUF:]
        gathered = jnp.dot(
            oh_ref[...], tab_ref[...], preferred_element_type=jnp.float32
        )
        for t in temps:
            t[...] = gathered
        copies = [
            pltpu.make_async_copy(temps[b % NBUF], o_ref.at[b], sems[b % NBUF])
            for b in range(batch)
        ]
        for c in copies:
            c.start()
        for c in copies:
            c.wait()

    return pl.pallas_call(
        body,
        in_specs=[
            pl.BlockSpec((seq, nrows), lambda: (0, 0)),
            pl.BlockSpec((nrows, D_MODEL), lambda: (0, 0)),
        ],
        out_specs=pl.BlockSpec(memory_space=pl.ANY),
        out_shape=jax.ShapeDtypeStruct((batch, seq, D_MODEL), jnp.float32),
        scratch_shapes=[pltpu.VMEM((seq, D_MODEL), jnp.float32)] * NBUF
        + [pltpu.SemaphoreType.DMA] * NBUF,
    )(jnp.asarray(onehot), jnp.asarray(table))
